# baseline (speedup 1.0000x reference)
import functools

import jax
import jax.numpy as jnp
from jax import lax
from jax.experimental import pallas as pl
from jax.experimental.pallas import tpu as pltpu
from jax.experimental.pallas import tpu_sc as plsc

NUM_USERS = 1000000
NUM_ITEMS = 1000000
FACTORS = 64
BATCH = 16384

NUM_CORES = 2
NUM_SUBCORES = 16
NW = NUM_CORES * NUM_SUBCORES

BPW = BATCH // NW
IDX_CHUNK = 128
NCHUNK = BPW // IDX_CHUNK


def _sc_body(users_hbm, items_hbm, gu_hbm, gi_hbm, bi_hbm,
             beta_hbm, gu_out_hbm, gi_out_hbm,
             uidx_v, iidx_v, gu_v, gi_v, bi_v,
             sem_u, sem_i, sem_b, sem_out):
    wid = lax.axis_index("s") * NUM_CORES + lax.axis_index("c")
    base = wid * BPW

    pltpu.sync_copy(users_hbm.at[pl.ds(wid * NCHUNK, NCHUNK)], uidx_v)
    pltpu.sync_copy(items_hbm.at[pl.ds(wid * NCHUNK, NCHUNK)], iidx_v)

    gu_cps = [pltpu.async_copy(gu_hbm.at[uidx_v.at[j]],
                               gu_v.at[pl.ds(j * IDX_CHUNK, IDX_CHUNK)], sem_u)
              for j in range(NCHUNK)]
    gi_cps = [pltpu.async_copy(gi_hbm.at[iidx_v.at[j]],
                               gi_v.at[pl.ds(j * IDX_CHUNK, IDX_CHUNK)], sem_i)
              for j in range(NCHUNK)]
    bi_cps = [pltpu.async_copy(bi_hbm.at[iidx_v.at[j]],
                               bi_v.at[pl.ds(j * IDX_CHUNK, IDX_CHUNK)], sem_b)
              for j in range(NCHUNK)]

    for cp in gu_cps:
        cp.wait()
    out_gu = pltpu.async_copy(gu_v, gu_out_hbm.at[pl.ds(base, BPW)], sem_out)
    for cp in gi_cps:
        cp.wait()
    out_gi = pltpu.async_copy(gi_v, gi_out_hbm.at[pl.ds(base, BPW)], sem_out)
    for cp in bi_cps:
        cp.wait()
    out_bi = pltpu.async_copy(bi_v, beta_hbm.at[pl.ds(base, BPW)], sem_out)
    out_gu.wait()
    out_gi.wait()
    out_bi.wait()


@functools.partial(
    pl.kernel,
    out_type=(
        jax.ShapeDtypeStruct((BATCH,), jnp.float32),
        jax.ShapeDtypeStruct((BATCH, FACTORS), jnp.float32),
        jax.ShapeDtypeStruct((BATCH, FACTORS), jnp.float32),
    ),
    mesh=plsc.VectorSubcoreMesh(core_axis_name="c", subcore_axis_name="s"),
    compiler_params=pltpu.CompilerParams(use_tc_tiling_on_sc=False),
    scratch_types=[
        pltpu.VMEM((NCHUNK, IDX_CHUNK), jnp.int32),
        pltpu.VMEM((NCHUNK, IDX_CHUNK), jnp.int32),
        pltpu.VMEM((BPW, FACTORS), jnp.float32),
        pltpu.VMEM((BPW, FACTORS), jnp.float32),
        pltpu.VMEM((BPW,), jnp.float32),
        pltpu.SemaphoreType.DMA,
        pltpu.SemaphoreType.DMA,
        pltpu.SemaphoreType.DMA,
        pltpu.SemaphoreType.DMA,
    ],
)
def _gather_sc(users_hbm, items_hbm, gu_hbm, gi_hbm, bi_hbm, *rest):
    _sc_body(users_hbm, items_hbm, gu_hbm, gi_hbm, bi_hbm, *rest)


TC_BLOCK = 2048


def _dot_tc_body(gu_ref, gi_ref, beta_ref, xui_ref):
    xui_ref[...] = beta_ref[...] + jnp.sum(gu_ref[...] * gi_ref[...], axis=1)


_dot_tc = pl.pallas_call(
    _dot_tc_body,
    grid=(BATCH // TC_BLOCK,),
    in_specs=[
        pl.BlockSpec((TC_BLOCK, FACTORS), lambda i: (i, 0)),
        pl.BlockSpec((TC_BLOCK, FACTORS), lambda i: (i, 0)),
        pl.BlockSpec((TC_BLOCK,), lambda i: (i,)),
    ],
    out_specs=pl.BlockSpec((TC_BLOCK,), lambda i: (i,)),
    out_shape=jax.ShapeDtypeStruct((BATCH,), jnp.float32),
)


def kernel(users_indices, items_indices, Gu, Gi, Bi):
    users2d = users_indices.astype(jnp.int32).reshape(NW * NCHUNK, IDX_CHUNK)
    items2d = items_indices.astype(jnp.int32).reshape(NW * NCHUNK, IDX_CHUNK)
    bi_flat = Bi.reshape(NUM_ITEMS)
    beta_i, gamma_u, gamma_i = _gather_sc(users2d, items2d, Gu, Gi, bi_flat)
    xui = _dot_tc(gamma_u, gamma_i, beta_i)
    return (xui, beta_i, gamma_u, gamma_i)
